# baseline (speedup 1.0000x reference)
import jax
import jax.numpy as jnp
from jax.experimental import pallas as pl
import jax.experimental.pallas.tpu as pltpu


def _lane_group_reduce(v, op):
    for s in (8, 16, 32, 64):
        v = op(v, pltpu.roll(v, s, 1))
    return v


def _onehot_body(x_ref, o_ref):
    xb = x_ref[0]
    colmax = jnp.max(xb, axis=0, keepdims=True)
    colmin = jnp.min(xb, axis=0, keepdims=True)
    colmax = _lane_group_reduce(colmax, jnp.maximum)
    colmin = _lane_group_reduce(colmin, jnp.minimum)

    r = jax.lax.broadcasted_iota(jnp.int32, xb.shape, 0)
    l = jax.lax.broadcasted_iota(jnp.int32, xb.shape, 1)
    cidx = r * 16 + (l >> 3)

    big = jnp.int32(1 << 30)
    idxm = jnp.where(xb == colmax, cidx, big)
    amin = jnp.min(idxm, axis=0, keepdims=True)
    amin = _lane_group_reduce(amin, jnp.minimum)

    valid = colmax != colmin
    one = (cidx == amin) & valid
    o_ref[0] = one.astype(jnp.float32)


def kernel(x):
    B, C, N = x.shape
    xr = x.reshape(B, C // 16, 16 * N)
    out = pl.pallas_call(
        _onehot_body,
        grid=(B,),
        in_specs=[pl.BlockSpec((1, C // 16, 16 * N), lambda b: (b, 0, 0))],
        out_specs=pl.BlockSpec((1, C // 16, 16 * N), lambda b: (b, 0, 0)),
        out_shape=jax.ShapeDtypeStruct((B, C // 16, 16 * N), jnp.float32),
    )(xr)
    return out.reshape(B, C, N)


# speedup vs baseline: 3.7999x; 3.7999x over previous
import jax
import jax.numpy as jnp
from jax.experimental import pallas as pl


def _onehot_body(x_ref, o_ref):
    xb = x_ref[0]
    colmax = jnp.max(xb, axis=1, keepdims=True)
    colmin = jnp.min(xb, axis=1, keepdims=True)

    cidx = jax.lax.broadcasted_iota(jnp.int32, xb.shape, 1)
    big = jnp.int32(1 << 30)
    idxm = jnp.where(xb == colmax, cidx, big)
    amin = jnp.min(idxm, axis=1, keepdims=True)

    valid = colmax != colmin
    one = (cidx == amin) & valid
    o_ref[0] = one.astype(jnp.float32)


def kernel(x):
    B, C, N = x.shape
    xt = jnp.transpose(x, (0, 2, 1))
    out = pl.pallas_call(
        _onehot_body,
        grid=(B,),
        in_specs=[pl.BlockSpec((1, N, C), lambda b: (b, 0, 0))],
        out_specs=pl.BlockSpec((1, N, C), lambda b: (b, 0, 0)),
        out_shape=jax.ShapeDtypeStruct((B, N, C), jnp.float32),
    )(xt)
    return jnp.transpose(out, (0, 2, 1))


# speedup vs baseline: 6.6272x; 1.7441x over previous
import jax
import jax.numpy as jnp
from jax.experimental import pallas as pl


_BB = 4


def _onehot_body(x_ref, o_ref):
    xb = x_ref[...]
    colmax = jnp.max(xb, axis=2, keepdims=True)
    colmin = jnp.min(xb, axis=2, keepdims=True)

    cidx = jax.lax.broadcasted_iota(jnp.int32, xb.shape, 2)
    big = jnp.int32(1 << 30)
    idxm = jnp.where(xb == colmax, cidx, big)
    amin = jnp.min(idxm, axis=2, keepdims=True)

    valid = colmax != colmin
    one = (cidx == amin) & valid
    o_ref[...] = one.astype(jnp.float32)


def kernel(x):
    B, C, N = x.shape
    xt = jnp.transpose(x, (0, 2, 1))
    out = pl.pallas_call(
        _onehot_body,
        grid=(B // _BB,),
        in_specs=[pl.BlockSpec((_BB, N, C), lambda b: (b, 0, 0))],
        out_specs=pl.BlockSpec((_BB, N, C), lambda b: (b, 0, 0)),
        out_shape=jax.ShapeDtypeStruct((B, N, C), jnp.float32),
    )(xt)
    return jnp.transpose(out, (0, 2, 1))


# speedup vs baseline: 7.1511x; 1.0791x over previous
import jax
import jax.numpy as jnp
from jax.experimental import pallas as pl


_BB = 8


def _onehot_body(x_ref, o_ref):
    xb = x_ref[...]
    colmax = jnp.max(xb, axis=2, keepdims=True)
    colmin = jnp.min(xb, axis=2, keepdims=True)

    cidx = jax.lax.broadcasted_iota(jnp.int32, xb.shape, 2)
    big = jnp.int32(1 << 30)
    idxm = jnp.where(xb == colmax, cidx, big)
    amin = jnp.min(idxm, axis=2, keepdims=True)

    valid = colmax != colmin
    one = (cidx == amin) & valid
    o_ref[...] = one.astype(jnp.float32)


def kernel(x):
    B, C, N = x.shape
    xt = jnp.transpose(x, (0, 2, 1))
    out = pl.pallas_call(
        _onehot_body,
        grid=(B // _BB,),
        in_specs=[pl.BlockSpec((_BB, N, C), lambda b: (b, 0, 0))],
        out_specs=pl.BlockSpec((_BB, N, C), lambda b: (b, 0, 0)),
        out_shape=jax.ShapeDtypeStruct((B, N, C), jnp.float32),
    )(xt)
    return jnp.transpose(out, (0, 2, 1))
